# speedup vs baseline: 2.3594x; 2.3594x over previous
import functools

import jax
import jax.numpy as jnp
from jax import lax
from jax.experimental import pallas as pl
from jax.experimental.pallas import tpu as pltpu
from jax.experimental.pallas import tpu_sc as plsc

N_NODES = 10000
K = 32
D_FEAT = 128
B = N_NODES * K
NW = 32
BPW = B // NW
CHUNK = 80
NCHUNK = BPW // CHUNK
LANES = 16


def _gather_rows(table, idx):
    mesh = plsc.VectorSubcoreMesh(core_axis_name="c", subcore_axis_name="s")

    @functools.partial(
        pl.kernel,
        mesh=mesh,
        out_type=jax.ShapeDtypeStruct((B, D_FEAT), jnp.float32),
        scratch_types=[
            pltpu.VMEM((BPW,), jnp.int32),
            pltpu.VMEM((CHUNK, D_FEAT), jnp.float32),
            pltpu.VMEM((CHUNK, D_FEAT), jnp.float32),
            pltpu.SemaphoreType.DMA,
            pltpu.SemaphoreType.DMA,
        ],
    )
    def k(table_hbm, idx_hbm, out_hbm, idx_v, buf0, buf1, gsem, wsem):
        nc = 2
        wid = lax.axis_index("s") * nc + lax.axis_index("c")
        base = pl.multiple_of(wid * BPW, 8)

        pltpu.sync_copy(idx_hbm.at[pl.ds(base, BPW)], idx_v)

        def fix(i, carry):
            off = pl.multiple_of(i * LANES, 8)
            v = idx_v[pl.ds(off, LANES)]
            idx_v[pl.ds(off, LANES)] = jnp.where(v < 0, N_NODES, v)
            return carry

        lax.fori_loop(0, BPW // LANES, fix, 0)

        def chunk(j, carry):
            off = pl.multiple_of(j * CHUNK, 8)
            pltpu.async_copy(
                table_hbm.at[idx_v.at[pl.ds(off, CHUNK)]], buf0, gsem
            ).wait()
            pltpu.async_copy(
                buf0, out_hbm.at[pl.ds(base + off, CHUNK)], wsem
            ).wait()
            return carry

        lax.fori_loop(0, NCHUNK, chunk, 0)

    return k(table, idx)


def kernel(features, nidx):
    table = jnp.concatenate(
        [features, jnp.zeros((1, D_FEAT), jnp.float32)], axis=0
    )
    idx = nidx.astype(jnp.int32).reshape(B)
    out = _gather_rows(table, idx)
    return out.reshape(N_NODES, K * D_FEAT)


# speedup vs baseline: 3.0750x; 1.3033x over previous
import functools

import jax
import jax.numpy as jnp
from jax import lax
from jax.experimental import pallas as pl
from jax.experimental.pallas import tpu as pltpu
from jax.experimental.pallas import tpu_sc as plsc

N_NODES = 10000
K = 32
D_FEAT = 128
B = N_NODES * K
NW = 32
BPW = B // NW
CHUNK = 80
NCHUNK = BPW // CHUNK
RING = 5
LANES = 16


def _gather_rows(table, idx):
    mesh = plsc.VectorSubcoreMesh(core_axis_name="c", subcore_axis_name="s")

    @functools.partial(
        pl.kernel,
        mesh=mesh,
        out_type=jax.ShapeDtypeStruct((B, D_FEAT), jnp.float32),
        scratch_types=[
            pltpu.VMEM((BPW,), jnp.int32),
        ]
        + [pltpu.VMEM((CHUNK, D_FEAT), jnp.float32) for _ in range(RING)]
        + [pltpu.SemaphoreType.DMA for _ in range(RING)],
    )
    def k(table_hbm, idx_hbm, out_hbm, idx_v, *rest):
        bufs = rest[:RING]
        sems = rest[RING:]
        nc = 2
        wid = lax.axis_index("s") * nc + lax.axis_index("c")
        base = pl.multiple_of(wid * BPW, 8)

        pltpu.sync_copy(idx_hbm.at[pl.ds(base, BPW)], idx_v)

        def fix_chunk(off):
            for i in range(CHUNK // LANES):
                o = pl.multiple_of(off + i * LANES, 8)
                v = idx_v[pl.ds(o, LANES)]
                idx_v[pl.ds(o, LANES)] = jnp.where(v < 0, N_NODES, v)

        def fire(off, b):
            pltpu.async_copy(
                table_hbm.at[idx_v.at[pl.ds(off, CHUNK)]], bufs[b], sems[b]
            )

        for b in range(RING):
            fix_chunk(b * CHUNK)
            fire(b * CHUNK, b)

        def round_(g, carry):
            for b in range(RING):
                j = g * RING + b
                off = pl.multiple_of(j * CHUNK, 8)
                pltpu.make_async_copy(
                    table_hbm.at[idx_v.at[pl.ds(off, CHUNK)]], bufs[b], sems[b]
                ).wait()
                pltpu.sync_copy(bufs[b], out_hbm.at[pl.ds(base + off, CHUNK)])

                @pl.when(j + RING < NCHUNK)
                def _():
                    noff = pl.multiple_of((j + RING) * CHUNK, 8)
                    fix_chunk(noff)
                    fire(noff, b)

            return carry

        lax.fori_loop(0, NCHUNK // RING, round_, 0)

    return k(table, idx)


def kernel(features, nidx):
    table = jnp.concatenate(
        [features, jnp.zeros((1, D_FEAT), jnp.float32)], axis=0
    )
    idx = nidx.astype(jnp.int32).reshape(B)
    out = _gather_rows(table, idx)
    return out.reshape(N_NODES, K * D_FEAT)
